# Initial kernel scaffold; baseline (speedup 1.0000x reference)
import jax
import jax.numpy as jnp
from jax.experimental import pallas as pl


def kernel(x, edge_index, W1, b1, W2, b2):
    raise NotImplementedError("write your pallas kernel here")


# speedup vs baseline: 26.8724x; 26.8724x over previous
import functools

import jax
import jax.numpy as jnp
from jax import lax
from jax.experimental import pallas as pl
from jax.experimental.pallas import tpu as pltpu
from jax.experimental.pallas import tpu_sc as plsc

N = 10000
E = 320000
F = 128
H = 128
C = 16

NC = 2
NS = 16
NW = NC * NS
EPT = E // NW
K = 125
NCHUNK = EPT // K
NPAD = 10240
RPT = NPAD // NS

_mesh = plsc.VectorSubcoreMesh(core_axis_name="c", subcore_axis_name="s")


def _deg_body(dst_hbm, ones_hbm, zeros_hbm, out_hbm, dstv, onesv, acc):
    c = lax.axis_index("c")
    s = lax.axis_index("s")
    wid = s * NC + c
    pltpu.sync_copy(zeros_hbm.at[pl.ds(s * RPT, RPT)], acc.at[pl.ds(s * RPT, RPT)])
    pltpu.sync_copy(ones_hbm, onesv)
    pltpu.sync_copy(dst_hbm.at[wid], dstv)
    plsc.subcore_barrier()

    @pl.loop(0, NCHUNK)
    def _(j):
        pltpu.sync_copy(onesv, acc.at[dstv.at[j]], add=True)

    plsc.subcore_barrier()
    pltpu.sync_copy(acc.at[pl.ds(s * RPT, RPT)], out_hbm.at[c, pl.ds(s * RPT, RPT)])


def _agg_body(Dw, h_hbm, src_hbm, dst_hbm, zeros_hbm, out_hbm, srcv, dstv, rows, acc):
    c = lax.axis_index("c")
    s = lax.axis_index("s")
    wid = s * NC + c
    pltpu.sync_copy(zeros_hbm.at[pl.ds(s * RPT, RPT)], acc.at[pl.ds(s * RPT, RPT)])
    pltpu.sync_copy(src_hbm.at[wid], srcv)
    pltpu.sync_copy(dst_hbm.at[wid], dstv)
    plsc.subcore_barrier()

    @pl.loop(0, NCHUNK)
    def _(j):
        pltpu.sync_copy(h_hbm.at[srcv.at[j]], rows)
        pltpu.sync_copy(rows, acc.at[dstv.at[j]], add=True)

    plsc.subcore_barrier()
    pltpu.sync_copy(acc.at[pl.ds(s * RPT, RPT)], out_hbm.at[c, pl.ds(s * RPT, RPT)])


def _make_deg():
    return pl.kernel(
        _deg_body,
        out_type=jax.ShapeDtypeStruct((NC, NPAD, 16), jnp.float32),
        mesh=_mesh,
        compiler_params=pltpu.CompilerParams(use_tc_tiling_on_sc=False),
        scratch_types=[
            pltpu.VMEM((NCHUNK, K), jnp.int32),
            pltpu.VMEM((K, 16), jnp.float32),
            pltpu.VMEM_SHARED((NPAD, 16), jnp.float32),
        ],
    )


def _make_agg(Dw):
    cp = pltpu.CompilerParams(use_tc_tiling_on_sc=False) if Dw < 128 else None
    return pl.kernel(
        functools.partial(_agg_body, Dw),
        out_type=jax.ShapeDtypeStruct((NC, NPAD, Dw), jnp.float32),
        mesh=_mesh,
        compiler_params=cp,
        scratch_types=[
            pltpu.VMEM((NCHUNK, K), jnp.int32),
            pltpu.VMEM((NCHUNK, K), jnp.int32),
            pltpu.VMEM((K, Dw), jnp.float32),
            pltpu.VMEM_SHARED((NPAD, Dw), jnp.float32),
        ],
    )


def _mm_body(x_ref, w_ref, o_ref):
    o_ref[...] = jnp.dot(x_ref[...], w_ref[...], preferred_element_type=jnp.float32)


def _norm_body(degp_ref, h1_ref, dinv_ref, h1s_ref):
    deg = degp_ref[0][:N, 0:1] + degp_ref[1][:N, 0:1] + 1.0
    dinv = lax.rsqrt(jnp.maximum(deg, 1.0))
    dinv_ref[...] = dinv
    h1s_ref[...] = h1_ref[...] * dinv


def _mid_body(agg_ref, h1s_ref, dinv_ref, b1_ref, w2_ref, h2s_ref):
    dinv = dinv_ref[...]
    agg = agg_ref[0][:N] + agg_ref[1][:N] + h1s_ref[...]
    z = jnp.maximum(agg * dinv + b1_ref[...], 0.0)
    h2s_ref[...] = jnp.dot(z * dinv, w2_ref[...], preferred_element_type=jnp.float32)


def _fin_body(agg_ref, h2s_ref, dinv_ref, b2_ref, out_ref):
    agg = agg_ref[0][:N] + agg_ref[1][:N] + h2s_ref[...]
    out_ref[...] = agg * dinv_ref[...] + b2_ref[...]


def kernel(x, edge_index, W1, b1, W2, b2):
    ei = edge_index.astype(jnp.int32)
    src3 = ei[0].reshape(NW, NCHUNK, K)
    dst3 = ei[1].reshape(NW, NCHUNK, K)
    ones_k = jnp.ones((K, 16), jnp.float32)
    zeros16 = jnp.zeros((NPAD, 16), jnp.float32)
    zeros128 = jnp.zeros((NPAD, H), jnp.float32)

    degp = _make_deg()(dst3, ones_k, zeros16)

    h1 = pl.pallas_call(
        _mm_body, out_shape=jax.ShapeDtypeStruct((N, H), jnp.float32)
    )(x, W1)

    dinv, h1s = pl.pallas_call(
        _norm_body,
        out_shape=(
            jax.ShapeDtypeStruct((N, 1), jnp.float32),
            jax.ShapeDtypeStruct((N, H), jnp.float32),
        ),
    )(degp, h1)

    agg1 = _make_agg(H)(h1s, src3, dst3, zeros128)

    h2s = pl.pallas_call(
        _mid_body, out_shape=jax.ShapeDtypeStruct((N, C), jnp.float32)
    )(agg1, h1s, dinv, b1.reshape(1, H), W2)

    agg2 = _make_agg(C)(h2s, src3, dst3, zeros16)

    out = pl.pallas_call(
        _fin_body, out_shape=jax.ShapeDtypeStruct((N, C), jnp.float32)
    )(agg2, h2s, dinv, b2.reshape(1, C))
    return out
